# baseline (speedup 1.0000x reference)
import functools

import jax
import jax.numpy as jnp
from jax import lax
from jax.experimental import pallas as pl
from jax.experimental.pallas import tpu as pltpu
from jax.experimental.pallas import tpu_sc as plsc

DIM = 64
N = 4096 * 200
NC = 2
NS = 16
NW = NC * NS
PER_W = N // NW
C = 1024
CHUNKS = PER_W // C


def _emb_lookup(table, idx):
    mesh = plsc.VectorSubcoreMesh(core_axis_name="c", subcore_axis_name="s")

    @functools.partial(
        pl.kernel,
        mesh=mesh,
        out_type=jax.ShapeDtypeStruct((N, DIM), jnp.float32),
        scratch_types=[
            pltpu.VMEM((C,), jnp.int32),
            pltpu.VMEM((C, DIM), jnp.float32),
            pltpu.SemaphoreType.DMA,
        ],
        compiler_params=pltpu.CompilerParams(use_tc_tiling_on_sc=False),
    )
    def k(table_hbm, idx_hbm, out_hbm, idx_v, rows_v, sem):
        wid = lax.axis_index("s") * NC + lax.axis_index("c")
        base = wid * PER_W

        def body(g, carry):
            off = base + g * C
            pltpu.sync_copy(idx_hbm.at[pl.ds(off, C)], idx_v)
            pltpu.async_copy(table_hbm.at[idx_v], rows_v, sem).wait()
            pltpu.sync_copy(rows_v, out_hbm.at[pl.ds(off, C)])
            return carry

        lax.fori_loop(0, CHUNKS, body, 0)

    return k(table, idx)


def kernel(inputs, table):
    idx = inputs.reshape(-1).astype(jnp.int32)
    out = _emb_lookup(table, idx)
    return out.reshape(inputs.shape + (DIM,))


# speedup vs baseline: 1.0129x; 1.0129x over previous
import functools

import jax
import jax.numpy as jnp
from jax import lax
from jax.experimental import pallas as pl
from jax.experimental.pallas import tpu as pltpu
from jax.experimental.pallas import tpu_sc as plsc

DIM = 64
N = 4096 * 200
NC = 2
NS = 16
NW = NC * NS
PER_W = N // NW
C = 320
CHUNKS = PER_W // C
NBUF = 4
STEPS = CHUNKS // NBUF


def _emb_lookup(table, idx):
    mesh = plsc.VectorSubcoreMesh(core_axis_name="c", subcore_axis_name="s")

    @functools.partial(
        pl.kernel,
        mesh=mesh,
        out_type=jax.ShapeDtypeStruct((N, DIM), jnp.float32),
        scratch_types=[
            pltpu.VMEM((PER_W,), jnp.int32),
            pltpu.VMEM((NBUF, C, DIM), jnp.float32),
            pltpu.SemaphoreType.DMA((NBUF,)),
            pltpu.SemaphoreType.DMA((NBUF,)),
        ],
        compiler_params=pltpu.CompilerParams(use_tc_tiling_on_sc=False),
    )
    def k(table_hbm, idx_hbm, out_hbm, idx_all, rows_v, gsem, ssem):
        wid = lax.axis_index("s") * NC + lax.axis_index("c")
        base = wid * PER_W
        pltpu.sync_copy(idx_hbm.at[pl.ds(base, PER_W)], idx_all)

        def fire(g, b):
            pltpu.async_copy(
                table_hbm.at[idx_all.at[pl.ds(g * C, C)]],
                rows_v.at[b], gsem.at[b])

        def wait_g(b):
            pltpu.make_async_copy(
                table_hbm.at[idx_all.at[pl.ds(0, C)]],
                rows_v.at[b], gsem.at[b]).wait()

        def store(g, b):
            pltpu.async_copy(
                rows_v.at[b], out_hbm.at[pl.ds(base + g * C, C)], ssem.at[b])

        def wait_s(b):
            pltpu.make_async_copy(
                rows_v.at[b], out_hbm.at[pl.ds(0, C)], ssem.at[b]).wait()

        for b in range(NBUF - 1):
            fire(b, b)

        def group(s, is_first, is_last):
            for bi in range(NBUF):
                g = s * NBUF + bi
                wait_g(bi)
                store(g, bi)
                bp = (bi - 1) % NBUF
                nxt = g + NBUF - 1
                if is_first and bi == 0:
                    fire(nxt, bp)
                elif is_last and bi > 0:
                    pass
                else:
                    wait_s(bp)
                    fire(nxt, bp)

        group(0, True, False)

        def body(s, carry):
            group(s, False, False)
            return carry

        lax.fori_loop(1, STEPS - 1, body, 0)
        group(STEPS - 1, False, True)

        for b in range(NBUF):
            wait_s(b)

    return k(table, idx)


def kernel(inputs, table):
    idx = inputs.reshape(-1).astype(jnp.int32)
    out = _emb_lookup(table, idx)
    return out.reshape(inputs.shape + (DIM,))


# speedup vs baseline: 1.2410x; 1.2252x over previous
import functools

import jax
import jax.numpy as jnp
from jax import lax
from jax.experimental import pallas as pl
from jax.experimental.pallas import tpu as pltpu
from jax.experimental.pallas import tpu_sc as plsc

DIM = 64
PDIM = 128
N = 4096 * 200
NC = 2
NS = 16
NW = NC * NS
PER_W = N // NW
C = 160
CHUNKS = PER_W // C
NBUF = 4
STEPS = CHUNKS // NBUF


def _emb_lookup(table, idx):
    mesh = plsc.VectorSubcoreMesh(core_axis_name="c", subcore_axis_name="s")

    @functools.partial(
        pl.kernel,
        mesh=mesh,
        out_type=jax.ShapeDtypeStruct((N, PDIM), jnp.float32),
        scratch_types=[
            pltpu.VMEM((PER_W,), jnp.int32),
            pltpu.VMEM((NBUF, C, PDIM), jnp.float32),
            pltpu.SemaphoreType.DMA((NBUF,)),
            pltpu.SemaphoreType.DMA((NBUF,)),
        ],
        compiler_params=pltpu.CompilerParams(use_tc_tiling_on_sc=False),
    )
    def k(table_hbm, idx_hbm, out_hbm, idx_all, rows_v, gsem, ssem):
        wid = lax.axis_index("s") * NC + lax.axis_index("c")
        base = wid * PER_W
        pltpu.sync_copy(idx_hbm.at[pl.ds(base, PER_W)], idx_all)

        def fire(g, b):
            pltpu.async_copy(
                table_hbm.at[idx_all.at[pl.ds(g * C, C)]],
                rows_v.at[b], gsem.at[b])

        def wait_g(b):
            pltpu.make_async_copy(
                table_hbm.at[idx_all.at[pl.ds(0, C)]],
                rows_v.at[b], gsem.at[b]).wait()

        def store(g, b):
            pltpu.async_copy(
                rows_v.at[b], out_hbm.at[pl.ds(base + g * C, C)], ssem.at[b])

        def wait_s(b):
            pltpu.make_async_copy(
                rows_v.at[b], out_hbm.at[pl.ds(0, C)], ssem.at[b]).wait()

        for b in range(NBUF - 1):
            fire(b, b)

        def group(s, is_first, is_last):
            for bi in range(NBUF):
                g = s * NBUF + bi
                wait_g(bi)
                store(g, bi)
                bp = (bi - 1) % NBUF
                nxt = g + NBUF - 1
                if is_first and bi == 0:
                    fire(nxt, bp)
                elif is_last and bi > 0:
                    pass
                else:
                    wait_s(bp)
                    fire(nxt, bp)

        group(0, True, False)

        def body(s, carry):
            group(s, False, False)
            return carry

        lax.fori_loop(1, STEPS - 1, body, 0)
        group(STEPS - 1, False, True)

        for b in range(NBUF):
            wait_s(b)

    return k(table, idx)


def kernel(inputs, table):
    idx = inputs.reshape(-1).astype(jnp.int32)
    table_p = jnp.pad(table, ((0, 0), (0, PDIM - DIM)))
    out_p = _emb_lookup(table_p, idx)
    return out_p[:, :DIM].reshape(inputs.shape + (DIM,))


# speedup vs baseline: 1.3525x; 1.0899x over previous
import functools

import jax
import jax.numpy as jnp
from jax import lax
from jax.experimental import pallas as pl
from jax.experimental.pallas import tpu as pltpu
from jax.experimental.pallas import tpu_sc as plsc

DIM = 64
PDIM = 128
N = 4096 * 200
NC = 2
NS = 16
NW = NC * NS
PER_W = N // NW
C = 320
CHUNKS = PER_W // C
NBUF = 4
STEPS = CHUNKS // NBUF


def _emb_lookup(table, idx):
    mesh = plsc.VectorSubcoreMesh(core_axis_name="c", subcore_axis_name="s")

    @functools.partial(
        pl.kernel,
        mesh=mesh,
        out_type=jax.ShapeDtypeStruct((N, PDIM), jnp.float32),
        scratch_types=[
            pltpu.VMEM((PER_W,), jnp.int32),
            pltpu.VMEM((NBUF, C, DIM), jnp.float32),
            pltpu.SemaphoreType.DMA((NBUF,)),
            pltpu.SemaphoreType.DMA((NBUF,)),
        ],
        compiler_params=pltpu.CompilerParams(use_tc_tiling_on_sc=False),
    )
    def k(table_hbm, idx_hbm, out_hbm, idx_all, rows_v, gsem, ssem):
        wid = lax.axis_index("s") * NC + lax.axis_index("c")
        base = wid * PER_W
        pltpu.sync_copy(idx_hbm.at[pl.ds(base, PER_W)], idx_all)

        def fire(g, b):
            pltpu.async_copy(
                table_hbm.at[idx_all.at[pl.ds(g * C, C)]],
                rows_v.at[b], gsem.at[b])

        def wait_g(b):
            pltpu.make_async_copy(
                table_hbm.at[idx_all.at[pl.ds(0, C)]],
                rows_v.at[b], gsem.at[b]).wait()

        def store(g, b):
            pltpu.async_copy(
                rows_v.at[b],
                out_hbm.at[pl.ds(base + g * C, C), pl.ds(0, DIM)],
                ssem.at[b])

        def wait_s(b):
            pltpu.make_async_copy(
                rows_v.at[b],
                out_hbm.at[pl.ds(0, C), pl.ds(0, DIM)],
                ssem.at[b]).wait()

        for b in range(NBUF - 1):
            fire(b, b)

        def group(s, is_first, is_last):
            for bi in range(NBUF):
                g = s * NBUF + bi
                wait_g(bi)
                store(g, bi)
                bp = (bi - 1) % NBUF
                nxt = g + NBUF - 1
                if is_first and bi == 0:
                    fire(nxt, bp)
                elif is_last and bi > 0:
                    pass
                else:
                    wait_s(bp)
                    fire(nxt, bp)

        group(0, True, False)

        def body(s, carry):
            group(s, False, False)
            return carry

        lax.fori_loop(1, STEPS - 1, body, 0)
        group(STEPS - 1, False, True)

        for b in range(NBUF):
            wait_s(b)

    return k(table, idx)


def kernel(inputs, table):
    idx = inputs.reshape(-1).astype(jnp.int32)
    out_p = _emb_lookup(table, idx)
    return out_p[:, :DIM].reshape(inputs.shape + (DIM,))
